# Initial kernel scaffold; baseline (speedup 1.0000x reference)
import functools
import math

import jax
import jax.numpy as jnp
from jax import lax
from jax.experimental import pallas as pl
from jax.experimental.pallas import tpu as pltpu
from jax.experimental.pallas import tpu_sc as plsc

_EMB = 32
_SCALE = math.sqrt(float(_EMB))
_LANES = 16
_CH = 1600
_RU = 8


@functools.lru_cache(maxsize=None)
def _build(B, V, D, nc, ns):
    nw = nc * ns
    b_per_w = B // nw
    n_chunks = b_per_w // _CH
    mesh = plsc.VectorSubcoreMesh(core_axis_name="c", subcore_axis_name="s")

    def body(idx_hbm, table_hbm, out_hbm, idx_v, rows_v, sem):
        wid = lax.axis_index("s") * nc + lax.axis_index("c")
        base = wid * b_per_w

        def chunk(ci, carry):
            cbase = base + ci * _CH
            pltpu.sync_copy(idx_hbm.at[pl.ds(cbase, _CH)], idx_v)
            pltpu.async_copy(table_hbm.at[idx_v], rows_v, sem).wait()

            def scale(g, c2):
                for j in range(_RU):
                    r = g * _RU + j
                    rows_v[r, pl.ds(0, _LANES)] = (
                        rows_v[r, pl.ds(0, _LANES)] * _SCALE
                    )
                    rows_v[r, pl.ds(_LANES, _LANES)] = (
                        rows_v[r, pl.ds(_LANES, _LANES)] * _SCALE
                    )
                return c2

            lax.fori_loop(0, _CH // _RU, scale, 0)
            pltpu.sync_copy(rows_v, out_hbm.at[pl.ds(cbase, _CH)])
            return carry

        lax.fori_loop(0, n_chunks, chunk, 0)

    return pl.kernel(
        body,
        mesh=mesh,
        out_type=jax.ShapeDtypeStruct((B, D), jnp.float32),
        scratch_types=[
            pltpu.VMEM((_CH,), jnp.int32),
            pltpu.VMEM((_CH, D), jnp.float32),
            pltpu.SemaphoreType.DMA,
        ],
    )


def kernel(tokens, table):
    n, s = tokens.shape
    V, D = table.shape
    B = n * s
    idx = tokens.reshape(B).astype(jnp.int32)
    info = plsc.get_sparse_core_info()
    out = _build(B, V, D, info.num_cores, info.num_subcores)(idx, table)
    return out.reshape(n, s, D)


# speedup vs baseline: 1.0230x; 1.0230x over previous
import functools
import math

import jax
import jax.numpy as jnp
from jax import lax
from jax.experimental import pallas as pl
from jax.experimental.pallas import tpu as pltpu
from jax.experimental.pallas import tpu_sc as plsc

_EMB = 32
_SCALE = math.sqrt(float(_EMB))
_LANES = 16
_CH = 1600
_RU = 8


@functools.lru_cache(maxsize=None)
def _build(B, V, D, nc, ns):
    nw = nc * ns
    b_per_w = B // nw
    n_chunks = b_per_w // _CH
    mesh = plsc.VectorSubcoreMesh(core_axis_name="c", subcore_axis_name="s")

    def body(idx_hbm, table_hbm, out_hbm, idx_v, rows_v, sem):
        wid = lax.axis_index("s") * nc + lax.axis_index("c")
        base = wid * b_per_w

        def chunk(ci, carry):
            cbase = base + ci * _CH
            pltpu.sync_copy(idx_hbm.at[pl.ds(cbase, _CH)], idx_v)
            pltpu.async_copy(table_hbm.at[idx_v], rows_v, sem).wait()

            def scale(g, c2):
                for j in range(_RU):
                    r = g * _RU + j
                    rows_v[r, pl.ds(0, _LANES)] = (
                        rows_v[r, pl.ds(0, _LANES)] * _SCALE
                    )
                    rows_v[r, pl.ds(_LANES, _LANES)] = (
                        rows_v[r, pl.ds(_LANES, _LANES)] * _SCALE
                    )
                return c2

            lax.fori_loop(0, _CH // _RU, scale, 0)
            pltpu.sync_copy(rows_v, out_hbm.at[pl.ds(cbase, _CH)])
            return carry

        lax.fori_loop(0, n_chunks, chunk, 0)

    return pl.kernel(
        body,
        mesh=mesh,
        compiler_params=pltpu.CompilerParams(use_tc_tiling_on_sc=False),
        out_type=jax.ShapeDtypeStruct((B, D), jnp.float32),
        scratch_types=[
            pltpu.VMEM((_CH,), jnp.int32),
            pltpu.VMEM((_CH, D), jnp.float32),
            pltpu.SemaphoreType.DMA,
        ],
    )


def kernel(tokens, table):
    n, s = tokens.shape
    V, D = table.shape
    B = n * s
    idx = tokens.reshape(B).astype(jnp.int32)
    info = plsc.get_sparse_core_info()
    out = _build(B, V, D, info.num_cores, info.num_subcores)(idx, table)
    return out.reshape(n, s, D)


# speedup vs baseline: 1.5679x; 1.5326x over previous
import functools
import math

import jax
import jax.numpy as jnp
from jax import lax
from jax.experimental import pallas as pl
from jax.experimental.pallas import tpu as pltpu
from jax.experimental.pallas import tpu_sc as plsc

_L = 16
_IC = 128
_NJ = 5
_PP = 37
_RU = 8


@functools.lru_cache(maxsize=None)
def _detile(N, S, nc, ns):
    nw = nc * ns
    i_per_w = N // nw
    nk = i_per_w // _IC
    mesh = plsc.VectorSubcoreMesh(core_axis_name="c", subcore_axis_name="s")

    def body(tokT_hbm, out_hbm, buf_v, sem):
        wid = lax.axis_index("s") * nc + lax.axis_index("c")
        i0 = wid * i_per_w
        stages = [
            pltpu.async_copy(
                tokT_hbm.at[j, pl.ds(i0, i_per_w)], buf_v.at[j], sem
            )
            for j in range(S)
        ]
        for c in stages:
            c.wait()
        outs = []
        for j in range(S):
            for k in range(nk):
                outs.append(
                    pltpu.async_copy(
                        buf_v.at[j, pl.ds(k * _IC, _IC)],
                        out_hbm.at[j * (N // _IC) + wid * nk + k],
                        sem,
                    )
                )
        for c in outs:
            c.wait()

    return pl.kernel(
        body,
        mesh=mesh,
        compiler_params=pltpu.CompilerParams(
            use_tc_tiling_on_sc=True, needs_layout_passes=False
        ),
        out_type=jax.ShapeDtypeStruct((S * (N // _IC), _IC), jnp.int32),
        scratch_types=[
            pltpu.VMEM((S, i_per_w), jnp.int32),
            pltpu.SemaphoreType.DMA,
        ],
    )


@functools.lru_cache(maxsize=None)
def _emb(N, S, V, D, nc, ns):
    nw = nc * ns
    i_per_w = N // nw
    nib = i_per_w // _IC
    njc = S // _NJ
    ndb = D // 8
    blk = 8 * _IC
    scale = math.sqrt(float(D))
    mesh = plsc.VectorSubcoreMesh(core_axis_name="c", subcore_axis_name="s")

    ch = _NJ * _IC
    n_ch = nib * njc

    def body(idxJ_hbm, table_hbm, out_hbm, idx_v, rows_v, skew_v, sb_v,
             isem, gsem, ssem):
        wid = lax.axis_index("s") * nc + lax.axis_index("c")
        ib0 = wid * nib
        lane = lax.iota(jnp.int32, _L)

        def coords(n):
            return ib0 + n // njc, (n % njc) * _NJ

        def idx_copies(n, p):
            ibg, j0 = coords(n)
            return [
                pltpu.make_async_copy(
                    idxJ_hbm.at[(j0 + jl) * (N // _IC) + ibg],
                    idx_v.at[p * _NJ + jl],
                    isem,
                )
                for jl in range(_NJ)
            ]

        def gather_copies(p):
            return [
                pltpu.make_async_copy(
                    table_hbm.at[idx_v.at[p * _NJ + jl]],
                    rows_v.at[pl.ds(p * ch + jl * _IC, _IC)],
                    gsem,
                )
                for jl in range(_NJ)
            ]

        def scatter_copy(n, p):
            ibg, j0 = coords(n)
            return pltpu.make_async_copy(
                sb_v.at[pl.ds(p * _NJ * ndb, _NJ * ndb)],
                out_hbm.at[pl.ds(j0 * ndb, _NJ * ndb), pl.ds(ibg * blk, blk)],
                ssem,
            )

        for c in idx_copies(0, 0):
            c.start()
        for c in idx_copies(0, 0):
            c.wait()
        for c in gather_copies(0):
            c.start()

        def step(n, carry):
            p = lax.rem(n, 2)
            q = 1 - p

            @pl.when(n + 1 < n_ch)
            def _():
                for c in idx_copies(n + 1, q):
                    c.start()
                for c in idx_copies(n + 1, q):
                    c.wait()

            for c in gather_copies(p):
                c.wait()

            @pl.when(n + 1 < n_ch)
            def _():
                for c in gather_copies(q):
                    c.start()

            @pl.when(n >= 2)
            def _():
                scatter_copy(n - 2, p).wait()

            rbase = p * ch
            sbase = p * _NJ * ndb

            def skew_rows(r8, c3):
                for u in range(_RU):
                    r = r8 * _RU + u
                    skew_v[r, pl.ds(0, _L)] = (
                        rows_v[rbase + r, pl.ds(0, _L)] * scale
                    )
                    skew_v[r, pl.ds(_L, _L)] = (
                        rows_v[rbase + r, pl.ds(_L, _L)] * scale
                    )
                return c3

            lax.fori_loop(0, ch // _RU, skew_rows, 0)

            for jl in range(_NJ):
                for db in range(ndb):
                    for dr in range(8):
                        cv = jnp.full((_L,), db * 8 + dr, jnp.int32)
                        for g in range(_IC // _L):
                            rv = lane + (jl * _IC + g * _L)
                            vec = plsc.load_gather(skew_v, [rv, cv])
                            sb_v[
                                sbase + jl * ndb + db,
                                pl.ds(dr * _IC + g * _L, _L),
                            ] = vec

            scatter_copy(n, p).start()
            return carry

        lax.fori_loop(0, n_ch, step, 0)

        scatter_copy(n_ch - 2, lax.rem(n_ch - 2, 2)).wait()
        scatter_copy(n_ch - 1, lax.rem(n_ch - 1, 2)).wait()

    return pl.kernel(
        body,
        mesh=mesh,
        compiler_params=pltpu.CompilerParams(
            use_tc_tiling_on_sc=False, needs_layout_passes=False
        ),
        out_type=jax.ShapeDtypeStruct((S * ndb, (N // _IC) * blk), jnp.float32),
        scratch_types=[
            pltpu.VMEM((2 * _NJ, _IC), jnp.int32),
            pltpu.VMEM((2 * ch, D), jnp.float32),
            pltpu.VMEM((ch, _PP), jnp.float32),
            pltpu.VMEM((2 * _NJ * ndb, blk), jnp.float32),
            pltpu.SemaphoreType.DMA,
            pltpu.SemaphoreType.DMA,
            pltpu.SemaphoreType.DMA,
        ],
    )


def kernel(tokens, table):
    N, S = tokens.shape
    V, D = table.shape
    ndb = D // 8
    tokT = tokens.astype(jnp.int32).T
    info = plsc.get_sparse_core_info()
    idxJ = _detile(N, S, info.num_cores, info.num_subcores)(tokT)
    o2 = _emb(N, S, V, D, info.num_cores, info.num_subcores)(idxJ, table)
    o5 = o2.reshape(S, ndb, N // _IC, 8, _IC)
    return o5.transpose(2, 4, 0, 1, 3).reshape(N, S, D)
